# baseline (speedup 1.0000x reference)
import jax
import jax.numpy as jnp
from jax import lax
from jax.experimental import pallas as pl
from jax.experimental.pallas import tpu as pltpu
from jax.experimental.pallas import tpu_sc as plsc

N = 524288
K = 16
NWD = N // 128

N_TC = 262144
N_SC = N - N_TC
TC_WD = N_TC // 128

NC = 2
NS = 16
L = 16
NW = NC * NS
ROWS_W = N_SC // NW
C = 1024
NCHUNK = ROWS_W // C
CW = C // 128

BW = 256


def _spline(xv, c, t0, t1, y0, y1, d0, d1):
    eq0 = c == 0
    eq1 = c == K
    interior = jnp.logical_not(jnp.logical_or(eq0, eq1))
    dt = t1 - t0
    dy = y1 - y0
    sl = dy / dt
    e = (xv - t0) / dt
    ome = 1.0 - e
    n0 = dy * (sl * e * e + d0 * e * ome)
    n1 = sl + (d1 + d0 - 2.0 * sl) * e * ome
    n1s = jnp.where(interior, n1, 1.0)
    p = y0 + n0 / n1s
    p = jnp.where(interior, p, xv)
    p = jnp.where(eq0, d0 * xv + (y0 - d0 * t0), p)
    p = jnp.where(eq1, d1 * xv + (y1 - d1 * t1), p)
    return p


def _tc_body(x_ref, t_ref, y_ref, d_ref, o_ref):
    x = x_ref[...]
    t = t_ref[...]
    y = y_ref[...]
    d = d_ref[...]
    xb = x[None, :, None, :]

    def red(a):
        s = a[0] + a[1]
        return jnp.sum(s, axis=1)

    c = red((xb > t).astype(jnp.float32)).astype(jnp.int32)
    eq0 = c == 0
    eq1 = c == K
    k0 = jnp.where(eq0, 0, jnp.where(eq1, K - 2, c - 1))
    k1 = jnp.where(eq0, 1, jnp.where(eq1, K - 1, c))

    j0 = lax.broadcasted_iota(jnp.int32, (2, BW, 8, 128), 0)
    j2 = lax.broadcasted_iota(jnp.int32, (2, BW, 8, 128), 2)
    jarr = j0 * 8 + j2

    m0 = jarr == k0[None, :, None, :]
    m1 = jarr == k1[None, :, None, :]

    def gath(a, m):
        return red(jnp.where(m, a, 0.0))

    o_ref[...] = _spline(x, c, gath(t, m0), gath(t, m1), gath(y, m0),
                         gath(y, m1), gath(d, m0), gath(d, m1))


def _sc_body(x_hbm, t_hbm, y_hbm, d_hbm, out_hbm,
             t_v0, y_v0, d_v0, x_v0, o_v0,
             t_v1, y_v1, d_v1, x_v1, o_v1,
             sem_in0, sem_in1, sem_out0, sem_out1):
    wid = lax.axis_index("s") * NC + lax.axis_index("c")
    lane = lax.broadcasted_iota(jnp.int32, (L,), 0)
    w_base = N_TC + wid * ROWS_W
    w_bw = TC_WD + wid * (ROWS_W // 128)

    bufs = [(t_v0, y_v0, d_v0, x_v0, o_v0, sem_in0, sem_out0),
            (t_v1, y_v1, d_v1, x_v1, o_v1, sem_in1, sem_out1)]

    def start_in(ci):
        t_v, y_v, d_v, x_v, _, sem_in, _ = bufs[ci & 1]
        base = pl.multiple_of(w_base + ci * C, C)
        bw = pl.multiple_of(w_bw + ci * CW, 8)
        src = lambda h: h.at[:, pl.ds(bw, CW), :, :]
        return [pltpu.async_copy(src(t_hbm), t_v, sem_in),
                pltpu.async_copy(src(y_hbm), y_v, sem_in),
                pltpu.async_copy(src(d_hbm), d_v, sem_in),
                pltpu.async_copy(x_hbm.at[pl.ds(base, C)], x_v, sem_in)]

    def compute(ci):
        t_v, y_v, d_v, x_v, o_v, _, _ = bufs[ci & 1]

        @plsc.parallel_loop(0, C // L, unroll=4)
        def _grp(g):
            rhigh = jnp.full((L,), g >> 3, jnp.int32)
            rlow = (g * L) % 128 + lane
            xv = x_v[pl.ds(g * L, L)]

            def gat(ref, j):
                return plsc.load_gather(ref, [j >> 3, rhigh, j & 7, rlow])

            c = jnp.zeros((L,), jnp.int32)
            for s in (8, 4, 2, 1):
                c = c + jnp.where(xv > gat(t_v, c + (s - 1)), s, 0)
            c = c + jnp.where(xv > gat(t_v, c), 1, 0)

            eq0 = c == 0
            eq1 = c == K
            k0 = jnp.where(eq0, 0, jnp.where(eq1, K - 2, c - 1))
            k1 = jnp.where(eq0, 1, jnp.where(eq1, K - 1, c))

            o_v[pl.ds(g * L, L)] = _spline(
                xv, c, gat(t_v, k0), gat(t_v, k1), gat(y_v, k0),
                gat(y_v, k1), gat(d_v, k0), gat(d_v, k1))

    in_descs = {}
    out_descs = {}
    in_descs[0] = start_in(0)
    for ci in range(NCHUNK):
        if ci + 1 < NCHUNK:
            in_descs[ci + 1] = start_in(ci + 1)
        for desc in in_descs.pop(ci):
            desc.wait()
        compute(ci)
        if ci >= 2:
            out_descs.pop(ci - 2).wait()
        o_v = bufs[ci & 1][4]
        sem_out = bufs[ci & 1][6]
        base = pl.multiple_of(w_base + ci * C, C)
        out_descs[ci] = pltpu.async_copy(
            o_v, out_hbm.at[pl.ds(base - N_TC, C)], sem_out)
    for ci in sorted(out_descs):
        out_descs.pop(ci).wait()


def _tileview(a):
    return a.T.reshape(K // 8, 8, NWD, 128).transpose(0, 2, 1, 3)


def kernel(x, t, y, d):
    xf = x.reshape(N)
    tv, yv, dv = _tileview(t), _tileview(y), _tileview(d)

    mesh = plsc.VectorSubcoreMesh(
        core_axis_name="c", subcore_axis_name="s", num_cores=NC, num_subcores=NS
    )
    tblv = pltpu.VMEM((K // 8, CW, 8, 128), jnp.float32)
    vecv = pltpu.VMEM((C,), jnp.float32)
    out_sc = pl.kernel(
        _sc_body,
        out_type=jax.ShapeDtypeStruct((N_SC,), jnp.float32),
        mesh=mesh,
        compiler_params=pltpu.CompilerParams(needs_layout_passes=False),
        scratch_types=[
            tblv, tblv, tblv, vecv, vecv,
            tblv, tblv, tblv, vecv, vecv,
            pltpu.SemaphoreType.DMA, pltpu.SemaphoreType.DMA,
            pltpu.SemaphoreType.DMA, pltpu.SemaphoreType.DMA,
        ],
    )(xf, tv, yv, dv)

    tbl = pl.BlockSpec((2, BW, 8, 128), lambda i: (0, i, 0, 0))
    vec = pl.BlockSpec((BW, 128), lambda i: (i, 0))
    out_tc = pl.pallas_call(
        _tc_body,
        grid=(TC_WD // BW,),
        in_specs=[vec, tbl, tbl, tbl],
        out_specs=vec,
        out_shape=jax.ShapeDtypeStruct((TC_WD, 128), jnp.float32),
    )(xf.reshape(NWD, 128), tv, yv, dv)

    return jnp.concatenate([out_tc.reshape(N_TC), out_sc])[:, None]


# speedup vs baseline: 1.0097x; 1.0097x over previous
import jax
import jax.numpy as jnp
from jax import lax
from jax.experimental import pallas as pl
from jax.experimental.pallas import tpu as pltpu
from jax.experimental.pallas import tpu_sc as plsc

N = 524288
K = 16
NWD = N // 128

N_TC = 262144
N_SC = N - N_TC
TC_WD = N_TC // 128

NC = 2
NS = 16
L = 16
NW = NC * NS
ROWS_W = N_SC // NW
C = 1024
NCHUNK = ROWS_W // C
CW = C // 128

BW = 512


def _spline(xv, c, t0, t1, y0, y1, d0, d1):
    eq0 = c == 0
    eq1 = c == K
    interior = jnp.logical_not(jnp.logical_or(eq0, eq1))
    dt = t1 - t0
    dy = y1 - y0
    sl = dy / dt
    e = (xv - t0) / dt
    ome = 1.0 - e
    n0 = dy * (sl * e * e + d0 * e * ome)
    n1 = sl + (d1 + d0 - 2.0 * sl) * e * ome
    n1s = jnp.where(interior, n1, 1.0)
    p = y0 + n0 / n1s
    p = jnp.where(interior, p, xv)
    p = jnp.where(eq0, d0 * xv + (y0 - d0 * t0), p)
    p = jnp.where(eq1, d1 * xv + (y1 - d1 * t1), p)
    return p


def _tc_body(x_ref, t_ref, y_ref, d_ref, o_ref):
    x = x_ref[...]
    t = t_ref[...]
    y = y_ref[...]
    d = d_ref[...]
    xb = x[None, :, None, :]

    def red(a):
        s = a[0] + a[1]
        return jnp.sum(s, axis=1)

    c = red((xb > t).astype(jnp.float32)).astype(jnp.int32)
    eq0 = c == 0
    eq1 = c == K
    k0 = jnp.where(eq0, 0, jnp.where(eq1, K - 2, c - 1))
    k1 = jnp.where(eq0, 1, jnp.where(eq1, K - 1, c))

    j0 = lax.broadcasted_iota(jnp.int32, (2, BW, 8, 128), 0)
    j2 = lax.broadcasted_iota(jnp.int32, (2, BW, 8, 128), 2)
    jarr = j0 * 8 + j2

    m0 = jarr == k0[None, :, None, :]
    m1 = jarr == k1[None, :, None, :]

    def gath(a, m):
        return red(jnp.where(m, a, 0.0))

    o_ref[...] = _spline(x, c, gath(t, m0), gath(t, m1), gath(y, m0),
                         gath(y, m1), gath(d, m0), gath(d, m1))


def _sc_body(x_hbm, t_hbm, y_hbm, d_hbm, out_hbm,
             t_v0, y_v0, d_v0, x_v0, o_v0,
             t_v1, y_v1, d_v1, x_v1, o_v1,
             sem_in0, sem_in1, sem_out0, sem_out1):
    wid = lax.axis_index("s") * NC + lax.axis_index("c")
    lane = lax.broadcasted_iota(jnp.int32, (L,), 0)
    w_base = N_TC + wid * ROWS_W
    w_bw = TC_WD + wid * (ROWS_W // 128)

    bufs = [(t_v0, y_v0, d_v0, x_v0, o_v0, sem_in0, sem_out0),
            (t_v1, y_v1, d_v1, x_v1, o_v1, sem_in1, sem_out1)]

    def start_in(ci):
        t_v, y_v, d_v, x_v, _, sem_in, _ = bufs[ci & 1]
        base = pl.multiple_of(w_base + ci * C, C)
        bw = pl.multiple_of(w_bw + ci * CW, 8)
        src = lambda h: h.at[:, pl.ds(bw, CW), :, :]
        return [pltpu.async_copy(src(t_hbm), t_v, sem_in),
                pltpu.async_copy(src(y_hbm), y_v, sem_in),
                pltpu.async_copy(src(d_hbm), d_v, sem_in),
                pltpu.async_copy(x_hbm.at[pl.ds(base, C)], x_v, sem_in)]

    def compute(ci):
        t_v, y_v, d_v, x_v, o_v, _, _ = bufs[ci & 1]

        @plsc.parallel_loop(0, C // L, unroll=4)
        def _grp(g):
            rhigh = jnp.full((L,), g >> 3, jnp.int32)
            rlow = (g * L) % 128 + lane
            xv = x_v[pl.ds(g * L, L)]

            def gat(ref, j):
                return plsc.load_gather(ref, [j >> 3, rhigh, j & 7, rlow])

            c = jnp.zeros((L,), jnp.int32)
            for s in (8, 4, 2, 1):
                c = c + jnp.where(xv > gat(t_v, c + (s - 1)), s, 0)
            c = c + jnp.where(xv > gat(t_v, c), 1, 0)

            eq0 = c == 0
            eq1 = c == K
            k0 = jnp.where(eq0, 0, jnp.where(eq1, K - 2, c - 1))
            k1 = jnp.where(eq0, 1, jnp.where(eq1, K - 1, c))

            o_v[pl.ds(g * L, L)] = _spline(
                xv, c, gat(t_v, k0), gat(t_v, k1), gat(y_v, k0),
                gat(y_v, k1), gat(d_v, k0), gat(d_v, k1))

    in_descs = {}
    out_descs = {}
    in_descs[0] = start_in(0)
    for ci in range(NCHUNK):
        if ci + 1 < NCHUNK:
            in_descs[ci + 1] = start_in(ci + 1)
        for desc in in_descs.pop(ci):
            desc.wait()
        compute(ci)
        if ci >= 2:
            out_descs.pop(ci - 2).wait()
        o_v = bufs[ci & 1][4]
        sem_out = bufs[ci & 1][6]
        base = pl.multiple_of(w_base + ci * C, C)
        out_descs[ci] = pltpu.async_copy(
            o_v, out_hbm.at[pl.ds(base - N_TC, C)], sem_out)
    for ci in sorted(out_descs):
        out_descs.pop(ci).wait()


def _tileview(a):
    return a.T.reshape(K // 8, 8, NWD, 128).transpose(0, 2, 1, 3)


def kernel(x, t, y, d):
    xf = x.reshape(N)
    tv, yv, dv = _tileview(t), _tileview(y), _tileview(d)

    mesh = plsc.VectorSubcoreMesh(
        core_axis_name="c", subcore_axis_name="s", num_cores=NC, num_subcores=NS
    )
    tblv = pltpu.VMEM((K // 8, CW, 8, 128), jnp.float32)
    vecv = pltpu.VMEM((C,), jnp.float32)
    out_sc = pl.kernel(
        _sc_body,
        out_type=jax.ShapeDtypeStruct((N_SC,), jnp.float32),
        mesh=mesh,
        compiler_params=pltpu.CompilerParams(needs_layout_passes=False),
        scratch_types=[
            tblv, tblv, tblv, vecv, vecv,
            tblv, tblv, tblv, vecv, vecv,
            pltpu.SemaphoreType.DMA, pltpu.SemaphoreType.DMA,
            pltpu.SemaphoreType.DMA, pltpu.SemaphoreType.DMA,
        ],
    )(xf, tv, yv, dv)

    tbl = pl.BlockSpec((2, BW, 8, 128), lambda i: (0, i, 0, 0))
    vec = pl.BlockSpec((BW, 128), lambda i: (i, 0))
    out_tc = pl.pallas_call(
        _tc_body,
        grid=(TC_WD // BW,),
        in_specs=[vec, tbl, tbl, tbl],
        out_specs=vec,
        out_shape=jax.ShapeDtypeStruct((TC_WD, 128), jnp.float32),
    )(xf.reshape(NWD, 128), tv, yv, dv)

    return jnp.concatenate([out_tc.reshape(N_TC), out_sc])[:, None]
